# baseline (speedup 1.0000x reference)
import jax
import jax.numpy as jnp
from jax.experimental import pallas as pl
from jax.experimental.pallas import tpu as pltpu

_CompilerParams = getattr(pltpu, "CompilerParams", None) or getattr(
    pltpu, "TPUCompilerParams"
)


def _prologue_kernel(x_ref, w0_ref, b0_ref, wg_ref, bg_ref, h_ref, gate_ref):
    xb = x_ref[...].astype(jnp.bfloat16)
    h = jnp.dot(
        xb, w0_ref[...].astype(jnp.bfloat16), preferred_element_type=jnp.float32
    ) + b0_ref[...]
    h_ref[...] = h.astype(jnp.bfloat16)
    g = jnp.dot(
        xb, wg_ref[...].astype(jnp.bfloat16), preferred_element_type=jnp.float32
    ) + bg_ref[...]
    gate_ref[...] = jax.nn.sigmoid(g)


def _spmm_kernel(adj_ref, h_ref, gate_ref, out_ref):
    a = adj_ref[...].astype(jnp.bfloat16)
    acc = jnp.dot(a, h_ref[...], preferred_element_type=jnp.float32)
    out_ref[...] = gate_ref[...] * acc


def kernel(x, adj_list, W0, b0, Wg0, bg0):
    n, d_in = x.shape
    d_out = W0.shape[1]
    adj = adj_list[0]

    h, gate = pl.pallas_call(
        _prologue_kernel,
        out_shape=(
            jax.ShapeDtypeStruct((n, d_out), jnp.bfloat16),
            jax.ShapeDtypeStruct((n, d_out), jnp.float32),
        ),
    )(x, W0, b0.reshape(1, d_out), Wg0, bg0.reshape(1, d_out))

    bm = 400
    out = pl.pallas_call(
        _spmm_kernel,
        grid=(n // bm,),
        in_specs=[
            pl.BlockSpec((bm, n), lambda i: (i, 0)),
            pl.BlockSpec((n, d_out), lambda i: (0, 0)),
            pl.BlockSpec((bm, d_out), lambda i: (i, 0)),
        ],
        out_specs=pl.BlockSpec((bm, d_out), lambda i: (i, 0)),
        out_shape=jax.ShapeDtypeStruct((n, d_out), jnp.float32),
        compiler_params=_CompilerParams(dimension_semantics=("parallel",)),
    )(adj, h, gate)
    return out


# speedup vs baseline: 1.0037x; 1.0037x over previous
import jax
import jax.numpy as jnp
from jax.experimental import pallas as pl
from jax.experimental.pallas import tpu as pltpu

_CompilerParams = getattr(pltpu, "CompilerParams", None) or getattr(
    pltpu, "TPUCompilerParams"
)


def _prologue_kernel(x_ref, w0_ref, b0_ref, wg_ref, bg_ref, h_ref, gate_ref):
    x = x_ref[...]
    h_ref[...] = jnp.dot(
        x, w0_ref[...], preferred_element_type=jnp.float32
    ) + b0_ref[...]
    g = jnp.dot(x, wg_ref[...], preferred_element_type=jnp.float32) + bg_ref[...]
    gate_ref[...] = jax.nn.sigmoid(g)


def _spmm_kernel(adj_ref, h_ref, gate_ref, out_ref):
    acc = jnp.dot(adj_ref[...], h_ref[...], preferred_element_type=jnp.float32)
    out_ref[...] = gate_ref[...] * acc


def kernel(x, adj_list, W0, b0, Wg0, bg0):
    n, d_in = x.shape
    d_out = W0.shape[1]
    adj = adj_list[0]

    h, gate = pl.pallas_call(
        _prologue_kernel,
        out_shape=(
            jax.ShapeDtypeStruct((n, d_out), jnp.float32),
            jax.ShapeDtypeStruct((n, d_out), jnp.float32),
        ),
    )(x, W0, b0.reshape(1, d_out), Wg0, bg0.reshape(1, d_out))

    bm = 400
    out = pl.pallas_call(
        _spmm_kernel,
        grid=(n // bm,),
        in_specs=[
            pl.BlockSpec((bm, n), lambda i: (i, 0)),
            pl.BlockSpec((n, d_out), lambda i: (0, 0)),
            pl.BlockSpec((bm, d_out), lambda i: (i, 0)),
        ],
        out_specs=pl.BlockSpec((bm, d_out), lambda i: (i, 0)),
        out_shape=jax.ShapeDtypeStruct((n, d_out), jnp.float32),
        compiler_params=_CompilerParams(dimension_semantics=("parallel",)),
    )(adj, h, gate)
    return out


# speedup vs baseline: 1.1037x; 1.0996x over previous
import jax
import jax.numpy as jnp
from jax.experimental import pallas as pl
from jax.experimental.pallas import tpu as pltpu

_CompilerParams = getattr(pltpu, "CompilerParams", None) or getattr(
    pltpu, "TPUCompilerParams"
)

_BM = 400


def _fused_kernel(x_ref, adj_ref, w0_ref, b0_ref, wg_ref, bg_ref, out_ref, h_ref):
    i = pl.program_id(0)

    @pl.when(i == 0)
    def _():
        h_ref[...] = (
            jnp.dot(x_ref[...], w0_ref[...], preferred_element_type=jnp.float32)
            + b0_ref[...]
        )

    acc = jnp.dot(adj_ref[...], h_ref[...], preferred_element_type=jnp.float32)
    xs = x_ref[pl.ds(i * _BM, _BM), :]
    gate = jax.nn.sigmoid(
        jnp.dot(xs, wg_ref[...], preferred_element_type=jnp.float32) + bg_ref[...]
    )
    out_ref[...] = gate * acc


def kernel(x, adj_list, W0, b0, Wg0, bg0):
    n, d_in = x.shape
    d_out = W0.shape[1]
    adj = adj_list[0]

    out = pl.pallas_call(
        _fused_kernel,
        grid=(n // _BM,),
        in_specs=[
            pl.BlockSpec((n, d_in), lambda i: (0, 0)),
            pl.BlockSpec((_BM, n), lambda i: (i, 0)),
            pl.BlockSpec((d_in, d_out), lambda i: (0, 0)),
            pl.BlockSpec((1, d_out), lambda i: (0, 0)),
            pl.BlockSpec((d_in, d_out), lambda i: (0, 0)),
            pl.BlockSpec((1, d_out), lambda i: (0, 0)),
        ],
        out_specs=pl.BlockSpec((_BM, d_out), lambda i: (i, 0)),
        out_shape=jax.ShapeDtypeStruct((n, d_out), jnp.float32),
        scratch_shapes=[pltpu.VMEM((n, d_out), jnp.float32)],
        compiler_params=_CompilerParams(dimension_semantics=("arbitrary",)),
    )(x, adj, W0, b0.reshape(1, d_out), Wg0, bg0.reshape(1, d_out))
    return out
